# baseline (speedup 1.0000x reference)
import jax
import jax.numpy as jnp
from jax.experimental import pallas as pl
from jax.experimental.pallas import tpu as pltpu

_B = 16384
_C = 1000
_CA = 896
_CR = _C - _CA
_BR = 2048
_NB = _B // _BR


def _copies(out_ref, scr_a, scr_r, sems, slot, step):
    rows = pl.ds(step * _BR, _BR)
    a = pltpu.make_async_copy(
        scr_a.at[slot],
        out_ref.at[rows, pl.ds(0, _CA)],
        sems.at[slot, 0],
    )
    b = pltpu.make_async_copy(
        scr_r.at[slot],
        out_ref.at[rows, pl.ds(_CA, _CR)],
        sems.at[slot, 1],
    )
    return a, b


def _onehot_block(idx_ref, out_ref, scr_a, scr_r, sems):
    i = pl.program_id(0)
    slot = jax.lax.rem(i, 2)

    @pl.when(i >= 2)
    def _wait_prev():
        a, b = _copies(out_ref, scr_a, scr_r, sems, slot, i - 2)
        a.wait()
        b.wait()

    idx = idx_ref[0, 0, :].reshape(_BR, 1)
    cols_a = jax.lax.broadcasted_iota(jnp.int32, (_BR, _CA), 1)
    scr_a[slot] = jnp.where(idx == cols_a, 1.0, 0.0)
    cols_r = jax.lax.broadcasted_iota(jnp.int32, (_BR, _CR), 1) + _CA
    scr_r[slot] = jnp.where(idx == cols_r, 1.0, 0.0)

    a, b = _copies(out_ref, scr_a, scr_r, sems, slot, i)
    a.start()
    b.start()

    @pl.when(i == _NB - 1)
    def _drain():
        a1, b1 = _copies(out_ref, scr_a, scr_r, sems, 1 - slot, i - 1)
        a1.wait()
        b1.wait()
        a2, b2 = _copies(out_ref, scr_a, scr_r, sems, slot, i)
        a2.wait()
        b2.wait()


def kernel(idxs):
    idxs3 = idxs.astype(jnp.int32).reshape(_NB, 1, _BR)
    return pl.pallas_call(
        _onehot_block,
        grid=(_NB,),
        in_specs=[pl.BlockSpec((1, 1, _BR), lambda i: (i, 0, 0))],
        out_specs=pl.BlockSpec(memory_space=pl.ANY),
        out_shape=jax.ShapeDtypeStruct((_B, _C), jnp.float32),
        scratch_shapes=[
            pltpu.VMEM((2, _BR, _CA), jnp.float32),
            pltpu.VMEM((2, _BR, _CR), jnp.float32),
            pltpu.SemaphoreType.DMA((2, 2)),
        ],
    )(idxs3)


# speedup vs baseline: 1.0064x; 1.0064x over previous
import jax
import jax.numpy as jnp
from jax.experimental import pallas as pl
from jax.experimental.pallas import tpu as pltpu

_B = 16384
_C = 1000
_CA = 896
_CR = _C - _CA
_NCH = _CA // 128
_BR = 2048
_NB = _B // _BR


def _copies(out_ref, scr_a, scr_r, sems, slot, step):
    rows = pl.ds(step * _BR, _BR)
    cps = []
    for k in range(_NCH):
        cps.append(pltpu.make_async_copy(
            scr_a.at[slot, :, pl.ds(k * 128, 128)],
            out_ref.at[rows, pl.ds(k * 128, 128)],
            sems.at[slot, k],
        ))
    cps.append(pltpu.make_async_copy(
        scr_r.at[slot],
        out_ref.at[rows, pl.ds(_CA, _CR)],
        sems.at[slot, _NCH],
    ))
    return cps


def _onehot_block(idx_ref, out_ref, scr_a, scr_r, sems):
    i = pl.program_id(0)
    slot = jax.lax.rem(i, 2)

    @pl.when(i >= 2)
    def _wait_prev():
        for c in _copies(out_ref, scr_a, scr_r, sems, slot, i - 2):
            c.wait()

    idx = idx_ref[0, 0, :].reshape(_BR, 1)
    cols_a = jax.lax.broadcasted_iota(jnp.int32, (_BR, _CA), 1)
    scr_a[slot] = jnp.where(idx == cols_a, 1.0, 0.0)
    cols_r = jax.lax.broadcasted_iota(jnp.int32, (_BR, _CR), 1) + _CA
    scr_r[slot] = jnp.where(idx == cols_r, 1.0, 0.0)

    for c in _copies(out_ref, scr_a, scr_r, sems, slot, i):
        c.start()

    @pl.when(i == _NB - 1)
    def _drain():
        for c in _copies(out_ref, scr_a, scr_r, sems, 1 - slot, i - 1):
            c.wait()
        for c in _copies(out_ref, scr_a, scr_r, sems, slot, i):
            c.wait()


def kernel(idxs):
    idxs3 = idxs.astype(jnp.int32).reshape(_NB, 1, _BR)
    return pl.pallas_call(
        _onehot_block,
        grid=(_NB,),
        in_specs=[pl.BlockSpec((1, 1, _BR), lambda i: (i, 0, 0))],
        out_specs=pl.BlockSpec(memory_space=pl.ANY),
        out_shape=jax.ShapeDtypeStruct((_B, _C), jnp.float32),
        scratch_shapes=[
            pltpu.VMEM((2, _BR, _CA), jnp.float32),
            pltpu.VMEM((2, _BR, _CR), jnp.float32),
            pltpu.SemaphoreType.DMA((2, _NCH + 1)),
        ],
    )(idxs3)
